# Initial kernel scaffold; baseline (speedup 1.0000x reference)
import jax
import jax.numpy as jnp
from jax.experimental import pallas as pl


def kernel(inputs, embed_weight):
    raise NotImplementedError("write your pallas kernel here")


# speedup vs baseline: 5.2114x; 5.2114x over previous
import functools

import jax
import jax.numpy as jnp
from jax import lax
from jax.experimental import pallas as pl
from jax.experimental.pallas import tpu as pltpu
from jax.experimental.pallas import tpu_sc as plsc

_ROWS, _COLS = 16384, 200
_N = _ROWS * _COLS
_NC, _NS = 2, 16
_NW = _NC * _NS
_PER_W = _N // _NW
_C = 6400
_ITERS = _PER_W // _C
_GROUPS = _C // 16

_mesh = plsc.VectorSubcoreMesh(core_axis_name="c", subcore_axis_name="s")


@functools.partial(
    pl.kernel,
    out_type=jax.ShapeDtypeStruct((_N * 4,), jnp.float32),
    mesh=_mesh,
    scratch_types=[
        pltpu.VMEM((_C,), jnp.int32),
        pltpu.VMEM((_C * 4,), jnp.float32),
        pltpu.VMEM((32,), jnp.float32),
    ],
    compiler_params=pltpu.CompilerParams(needs_layout_passes=False),
)
def _lookup(idx_hbm, tab_hbm, out_hbm, idx_v, out_v, tab_v):
    wid = lax.axis_index("s") * _NC + lax.axis_index("c")
    base = wid * _PER_W
    pltpu.sync_copy(tab_hbm, tab_v)
    lane4 = lax.iota(jnp.int32, 16) * 4

    def chunk_body(it, carry):
        start = base + it * _C
        pltpu.sync_copy(idx_hbm.at[pl.ds(start, _C)], idx_v)

        def group_body(g, carry2):
            iv = idx_v[pl.ds(g * 16, 16)]
            obase = lane4 + g * 64
            for j in range(4):
                col = plsc.load_gather(tab_v, [iv + (6 * j)])
                plsc.store_scatter(out_v, [obase + j], col)
            return carry2

        lax.fori_loop(0, _GROUPS, group_body, 0)
        pltpu.sync_copy(out_v, out_hbm.at[pl.ds(start * 4, _C * 4)])
        return carry

    lax.fori_loop(0, _ITERS, chunk_body, 0)


def kernel(inputs, embed_weight):
    idx_flat = inputs.reshape(-1).astype(jnp.int32)
    tab = jnp.zeros((32,), jnp.float32)
    tab = tab.at[:24].set(embed_weight.astype(jnp.float32).T.reshape(-1))
    out_flat = _lookup(idx_flat, tab)
    return out_flat.reshape(_ROWS, _COLS, 4)


# speedup vs baseline: 67.6923x; 12.9893x over previous
import functools

import jax
import jax.numpy as jnp
from jax import lax
from jax.experimental import pallas as pl
from jax.experimental.pallas import tpu as pltpu
from jax.experimental.pallas import tpu_sc as plsc

_ROWS, _COLS = 16384, 200
_N = _ROWS * _COLS
_NC, _NS = 2, 16
_NW = _NC * _NS
_CB = _COLS // 8
_RB = _ROWS // 128
_RB_W = _RB // _NW

_mesh = plsc.VectorSubcoreMesh(core_axis_name="c", subcore_axis_name="s")


@functools.partial(
    pl.kernel,
    out_type=jax.ShapeDtypeStruct((_N * 4,), jnp.float32),
    mesh=_mesh,
    scratch_types=[
        pltpu.VMEM((_RB_W * 8 * 128,), jnp.int32),
        pltpu.VMEM((8, _RB_W * 4 * 128), jnp.float32),
        pltpu.VMEM((32,), jnp.float32),
    ],
    compiler_params=pltpu.CompilerParams(needs_layout_passes=False),
)
def _lookup(idx_hbm, tab_hbm, out_hbm, in_v, out_v, tab_v):
    wid = lax.axis_index("s") * _NC + lax.axis_index("c")
    rb0 = wid * _RB_W
    pltpu.sync_copy(tab_hbm, tab_v)

    def cb_body(cb, carry):
        in_off = (cb * _RB + rb0) * 1024
        pltpu.sync_copy(idx_hbm.at[pl.ds(in_off, _RB_W * 1024)], in_v)

        def row_body(i, carry2):
            cs = i // _RB_W
            rbl = i - cs * _RB_W
            ibase = rbl * 1024 + cs * 128
            obase = rbl * 512
            for g in range(8):
                iv = in_v[pl.ds(ibase + g * 16, 16)]
                for j in range(4):
                    col = plsc.load_gather(tab_v, [iv + (6 * j)])
                    out_v[cs, pl.ds(obase + j * 128 + g * 16, 16)] = col
            return carry2

        lax.fori_loop(0, 8 * _RB_W, row_body, 0)
        for cs in range(8):
            out_off = ((cb * 8 + cs) * _RB + rb0) * 512
            pltpu.sync_copy(out_v.at[cs], out_hbm.at[pl.ds(out_off, _RB_W * 512)])
        return carry

    lax.fori_loop(0, _CB, cb_body, 0)


def kernel(inputs, embed_weight):
    idx_lin = (
        inputs.astype(jnp.int32)
        .reshape(_RB, 128, _CB, 8)
        .transpose(2, 0, 3, 1)
        .reshape(-1)
    )
    tab = jnp.zeros((32,), jnp.float32)
    tab = tab.at[:24].set(embed_weight.astype(jnp.float32).T.reshape(-1))
    out_flat = _lookup(idx_lin, tab)
    return (
        out_flat.reshape(_COLS, _RB, 4, 128)
        .transpose(1, 3, 0, 2)
        .reshape(_ROWS, _COLS, 4)
    )


# speedup vs baseline: 81.4666x; 1.2035x over previous
import functools

import jax
import jax.numpy as jnp
from jax import lax
from jax.experimental import pallas as pl
from jax.experimental.pallas import tpu as pltpu
from jax.experimental.pallas import tpu_sc as plsc

_ROWS, _COLS = 16384, 200
_N = _ROWS * _COLS
_NC, _NS = 2, 16
_NW = _NC * _NS
_CB = _COLS // 8
_RB = _ROWS // 128
_RB_W = _RB // _NW

_mesh = plsc.VectorSubcoreMesh(core_axis_name="c", subcore_axis_name="s")


@functools.partial(
    pl.kernel,
    out_type=jax.ShapeDtypeStruct((_N * 4,), jnp.float32),
    mesh=_mesh,
    scratch_types=[
        pltpu.VMEM((2, _RB_W * 8 * 128), jnp.int32),
        pltpu.VMEM((2, 8, _RB_W * 4 * 128), jnp.float32),
        pltpu.VMEM((32,), jnp.float32),
        pltpu.SemaphoreType.DMA,
        pltpu.SemaphoreType.DMA,
        pltpu.SemaphoreType.DMA,
        pltpu.SemaphoreType.DMA,
    ],
    compiler_params=pltpu.CompilerParams(needs_layout_passes=False),
)
def _lookup(idx_hbm, tab_hbm, out_hbm, in_v, out_v, tab_v,
            isem0, isem1, osem0, osem1):
    wid = lax.axis_index("s") * _NC + lax.axis_index("c")
    rb0 = wid * _RB_W
    pltpu.sync_copy(tab_hbm, tab_v)
    isems = (isem0, isem1)
    osems = (osem0, osem1)

    def in_copy(cb, b):
        return pltpu.make_async_copy(
            idx_hbm.at[pl.ds((cb * _RB + rb0) * 1024, _RB_W * 1024)],
            in_v.at[b], isems[b])

    def out_copy(cb, b, cs):
        return pltpu.make_async_copy(
            out_v.at[b, cs],
            out_hbm.at[pl.ds(((cb * 8 + cs) * _RB + rb0) * 512, _RB_W * 512)],
            osems[b])

    def compute(b):
        def row_body(i, carry):
            cs = i // _RB_W
            rbl = i - cs * _RB_W
            ibase = rbl * 1024 + cs * 128
            obase = rbl * 512
            for g in range(8):
                iv = in_v[b, pl.ds(ibase + g * 16, 16)]
                for j in range(4):
                    col = plsc.load_gather(tab_v, [iv + (6 * j)])
                    out_v[b, cs, pl.ds(obase + j * 128 + g * 16, 16)] = col
            return carry

        lax.fori_loop(0, 8 * _RB_W, row_body, 0)

    def unit(cb, b, skip_out_wait):
        in_copy(cb, b).wait()

        @pl.when(jnp.logical_not(skip_out_wait))
        def _():
            for cs in range(8):
                out_copy(cb, b, cs).wait()

        compute(b)
        for cs in range(8):
            out_copy(cb, b, cs).start()

    in_copy(0, 0).start()

    def pair_body(p, carry):
        cb0 = 2 * p
        in_copy(cb0 + 1, 1).start()
        unit(cb0, 0, p == 0)
        in_copy(cb0 + 2, 0).start()
        unit(cb0 + 1, 1, p == 0)
        return carry

    lax.fori_loop(0, (_CB - 1) // 2, pair_body, 0)
    unit(_CB - 1, 0, False)
    for cs in range(8):
        out_copy(_CB - 1, 0, cs).wait()
        out_copy(_CB - 2, 1, cs).wait()


def kernel(inputs, embed_weight):
    idx_lin = (
        inputs.astype(jnp.int32)
        .reshape(_RB, 128, _CB, 8)
        .transpose(2, 0, 3, 1)
        .reshape(-1)
    )
    tab = jnp.zeros((32,), jnp.float32)
    tab = tab.at[:24].set(embed_weight.astype(jnp.float32).T.reshape(-1))
    out_flat = _lookup(idx_lin, tab)
    return (
        out_flat.reshape(_COLS, _RB, 4, 128)
        .transpose(1, 3, 0, 2)
        .reshape(_ROWS, _COLS, 4)
    )


# speedup vs baseline: 181.7432x; 2.2309x over previous
import functools

import jax
import jax.numpy as jnp
from jax import lax
from jax.experimental import pallas as pl
from jax.experimental.pallas import tpu as pltpu
from jax.experimental.pallas import tpu_sc as plsc

_ROWS, _COLS = 16384, 200
_N = _ROWS * _COLS
_NC, _NS = 2, 16
_NW = _NC * _NS
_CB = _COLS // 8
_RB = _ROWS // 128
_RB_W = _RB // _NW

_mesh = plsc.VectorSubcoreMesh(core_axis_name="c", subcore_axis_name="s")


@functools.partial(
    pl.kernel,
    out_type=jax.ShapeDtypeStruct((_N * 4,), jnp.float32),
    mesh=_mesh,
    scratch_types=[
        pltpu.VMEM((2, _RB_W * 8 * 128), jnp.int32),
        pltpu.VMEM((2, 8, _RB_W * 4 * 128), jnp.float32),
        pltpu.VMEM((32,), jnp.float32),
        pltpu.SemaphoreType.DMA,
        pltpu.SemaphoreType.DMA,
        pltpu.SemaphoreType.DMA,
        pltpu.SemaphoreType.DMA,
    ],
    compiler_params=pltpu.CompilerParams(needs_layout_passes=False),
)
def _lookup(idx_hbm, tab_hbm, out_hbm, in_v, out_v, tab_v,
            isem0, isem1, osem0, osem1):
    wid = lax.axis_index("s") * _NC + lax.axis_index("c")
    rb0 = wid * _RB_W
    pltpu.sync_copy(tab_hbm, tab_v)
    isems = (isem0, isem1)
    osems = (osem0, osem1)

    def in_copy(cb, b):
        return pltpu.make_async_copy(
            idx_hbm.at[pl.ds((cb * _RB + rb0) * 1024, _RB_W * 1024)],
            in_v.at[b], isems[b])

    def out_copy(cb, b, cs):
        return pltpu.make_async_copy(
            out_v.at[b, cs],
            out_hbm.at[pl.ds(((cb * 8 + cs) * _RB + rb0) * 512, _RB_W * 512)],
            osems[b])

    def compute(b):
        @plsc.parallel_loop(0, 8 * _RB_W, unroll=2)
        def row_body(i):
            cs = i // _RB_W
            rbl = i - cs * _RB_W
            ibase = rbl * 1024 + cs * 128
            obase = rbl * 512
            for g in range(8):
                iv = in_v[b, pl.ds(ibase + g * 16, 16)]
                for j in range(4):
                    col = plsc.load_gather(tab_v, [iv + (6 * j)])
                    out_v[b, cs, pl.ds(obase + j * 128 + g * 16, 16)] = col

    def unit(cb, b, skip_out_wait):
        in_copy(cb, b).wait()

        @pl.when(jnp.logical_not(skip_out_wait))
        def _():
            for cs in range(8):
                out_copy(cb, b, cs).wait()

        compute(b)
        for cs in range(8):
            out_copy(cb, b, cs).start()

    in_copy(0, 0).start()

    def pair_body(p, carry):
        cb0 = 2 * p
        in_copy(cb0 + 1, 1).start()
        unit(cb0, 0, p == 0)
        in_copy(cb0 + 2, 0).start()
        unit(cb0 + 1, 1, p == 0)
        return carry

    lax.fori_loop(0, (_CB - 1) // 2, pair_body, 0)
    unit(_CB - 1, 0, False)
    for cs in range(8):
        out_copy(_CB - 1, 0, cs).wait()
        out_copy(_CB - 2, 1, cs).wait()


def kernel(inputs, embed_weight):
    idx_lin = (
        inputs.astype(jnp.int32)
        .reshape(_RB, 128, _CB, 8)
        .transpose(2, 0, 3, 1)
        .reshape(-1)
    )
    tab = jnp.zeros((32,), jnp.float32)
    tab = tab.at[:24].set(embed_weight.astype(jnp.float32).T.reshape(-1))
    out_flat = _lookup(idx_lin, tab)
    return (
        out_flat.reshape(_COLS, _RB, 4, 128)
        .transpose(1, 3, 0, 2)
        .reshape(_ROWS, _COLS, 4)
    )


# speedup vs baseline: 186.0465x; 1.0237x over previous
import functools

import jax
import jax.numpy as jnp
from jax import lax
from jax.experimental import pallas as pl
from jax.experimental.pallas import tpu as pltpu
from jax.experimental.pallas import tpu_sc as plsc

_ROWS, _COLS = 16384, 200
_N = _ROWS * _COLS
_NC, _NS = 2, 16
_NW = _NC * _NS
_CB = _COLS // 8
_RB = _ROWS // 128
_RB_W = _RB // _NW

_mesh = plsc.VectorSubcoreMesh(core_axis_name="c", subcore_axis_name="s")


@functools.partial(
    pl.kernel,
    out_type=jax.ShapeDtypeStruct((_N * 4,), jnp.float32),
    mesh=_mesh,
    scratch_types=[
        pltpu.VMEM((2, _RB_W * 8 * 128), jnp.int32),
        pltpu.VMEM((2, 8, _RB_W * 4 * 128), jnp.float32),
        pltpu.VMEM((32,), jnp.float32),
        pltpu.SemaphoreType.DMA,
        pltpu.SemaphoreType.DMA,
        pltpu.SemaphoreType.DMA,
        pltpu.SemaphoreType.DMA,
    ],
    compiler_params=pltpu.CompilerParams(needs_layout_passes=False),
)
def _lookup(idx_hbm, tab_hbm, out_hbm, in_v, out_v, tab_v,
            isem0, isem1, osem0, osem1):
    wid = lax.axis_index("s") * _NC + lax.axis_index("c")
    rb0 = wid * _RB_W
    pltpu.sync_copy(tab_hbm, tab_v)
    isems = (isem0, isem1)
    osems = (osem0, osem1)

    def in_copy(cb, b):
        return pltpu.make_async_copy(
            idx_hbm.at[pl.ds((cb * _RB + rb0) * 1024, _RB_W * 1024)],
            in_v.at[b], isems[b])

    def out_copy(cb, b, cs):
        return pltpu.make_async_copy(
            out_v.at[b, cs],
            out_hbm.at[pl.ds(((cb * 8 + cs) * _RB + rb0) * 512, _RB_W * 512)],
            osems[b])

    def compute(b):
        @plsc.parallel_loop(0, 8 * _RB_W, unroll=4)
        def row_body(i):
            cs = i // _RB_W
            rbl = i - cs * _RB_W
            ibase = rbl * 1024 + cs * 128
            obase = rbl * 512
            for g in range(8):
                iv = in_v[b, pl.ds(ibase + g * 16, 16)]
                for j in range(4):
                    col = plsc.load_gather(tab_v, [iv + (6 * j)])
                    out_v[b, cs, pl.ds(obase + j * 128 + g * 16, 16)] = col

    def unit(cb, b, skip_out_wait):
        in_copy(cb, b).wait()

        @pl.when(jnp.logical_not(skip_out_wait))
        def _():
            for cs in range(8):
                out_copy(cb, b, cs).wait()

        compute(b)
        for cs in range(8):
            out_copy(cb, b, cs).start()

    in_copy(0, 0).start()

    def pair_body(p, carry):
        cb0 = 2 * p
        in_copy(cb0 + 1, 1).start()
        unit(cb0, 0, p == 0)
        in_copy(cb0 + 2, 0).start()
        unit(cb0 + 1, 1, p == 0)
        return carry

    lax.fori_loop(0, (_CB - 1) // 2, pair_body, 0)
    unit(_CB - 1, 0, False)
    for cs in range(8):
        out_copy(_CB - 1, 0, cs).wait()
        out_copy(_CB - 2, 1, cs).wait()


def kernel(inputs, embed_weight):
    idx_lin = (
        inputs.astype(jnp.int32)
        .reshape(_RB, 128, _CB, 8)
        .transpose(2, 0, 3, 1)
        .reshape(-1)
    )
    tab = jnp.zeros((32,), jnp.float32)
    tab = tab.at[:24].set(embed_weight.astype(jnp.float32).T.reshape(-1))
    out_flat = _lookup(idx_lin, tab)
    return (
        out_flat.reshape(_COLS, _RB, 4, 128)
        .transpose(1, 3, 0, 2)
        .reshape(_ROWS, _COLS, 4)
    )


# speedup vs baseline: 243.5607x; 1.3091x over previous
import functools

import jax
import jax.numpy as jnp
from jax import lax
from jax.experimental import pallas as pl
from jax.experimental.pallas import tpu as pltpu
from jax.experimental.pallas import tpu_sc as plsc

_ROWS, _COLS = 16384, 200
_N = _ROWS * _COLS
_NC, _NS = 2, 16
_NW = _NC * _NS
_CB = _COLS // 8
_RB = _ROWS // 128
_RB_W = _RB // _NW

_mesh = plsc.VectorSubcoreMesh(core_axis_name="c", subcore_axis_name="s")


@functools.partial(
    pl.kernel,
    out_type=jax.ShapeDtypeStruct((_COLS, _RB * 4, 128), jnp.float32),
    mesh=_mesh,
    scratch_types=[
        pltpu.VMEM((2, _RB_W, 8, 128), jnp.int32),
        pltpu.VMEM((2, 8, _RB_W * 4, 128), jnp.float32),
        pltpu.VMEM((32,), jnp.float32),
        pltpu.SemaphoreType.DMA,
        pltpu.SemaphoreType.DMA,
        pltpu.SemaphoreType.DMA,
        pltpu.SemaphoreType.DMA,
    ],
    compiler_params=pltpu.CompilerParams(needs_layout_passes=False),
)
def _lookup(idx_hbm, tab_hbm, out_hbm, in_v, out_v, tab_v,
            isem0, isem1, osem0, osem1):
    wid = lax.axis_index("s") * _NC + lax.axis_index("c")
    rb0 = wid * _RB_W
    pltpu.sync_copy(tab_hbm, tab_v)
    isems = (isem0, isem1)
    osems = (osem0, osem1)

    def in_copy(cb, b):
        return pltpu.make_async_copy(
            idx_hbm.at[cb, pl.ds(rb0, _RB_W)], in_v.at[b], isems[b])

    def out_copy(cb, b, cs):
        return pltpu.make_async_copy(
            out_v.at[b, cs],
            out_hbm.at[cb * 8 + cs, pl.ds(rb0 * 4, _RB_W * 4)],
            osems[b])

    def compute(b):
        @plsc.parallel_loop(0, 8 * _RB_W, unroll=4)
        def row_body(i):
            cs = i // _RB_W
            rbl = i - cs * _RB_W
            for g in range(8):
                iv = in_v[b, rbl, cs, pl.ds(g * 16, 16)]
                for j in range(4):
                    col = plsc.load_gather(tab_v, [iv + (6 * j)])
                    out_v[b, cs, rbl * 4 + j, pl.ds(g * 16, 16)] = col

    def unit(cb, b, skip_out_wait):
        in_copy(cb, b).wait()

        @pl.when(jnp.logical_not(skip_out_wait))
        def _():
            for cs in range(8):
                out_copy(cb, b, cs).wait()

        compute(b)
        for cs in range(8):
            out_copy(cb, b, cs).start()

    in_copy(0, 0).start()

    def pair_body(p, carry):
        cb0 = 2 * p
        in_copy(cb0 + 1, 1).start()
        unit(cb0, 0, p == 0)
        in_copy(cb0 + 2, 0).start()
        unit(cb0 + 1, 1, p == 0)
        return carry

    lax.fori_loop(0, (_CB - 1) // 2, pair_body, 0)
    unit(_CB - 1, 0, False)
    for cs in range(8):
        out_copy(_CB - 1, 0, cs).wait()
        out_copy(_CB - 2, 1, cs).wait()


def kernel(inputs, embed_weight):
    idx4 = (
        inputs.astype(jnp.int32)
        .reshape(_RB, 128, _CB, 8)
        .transpose(2, 0, 3, 1)
    )
    tab = jnp.zeros((32,), jnp.float32)
    tab = tab.at[:24].set(embed_weight.astype(jnp.float32).T.reshape(-1))
    out3 = _lookup(idx4, tab)
    return (
        out3.reshape(_COLS, _RB, 4, 128)
        .transpose(1, 3, 0, 2)
        .reshape(_ROWS, _COLS, 4)
    )
